# baseline (speedup 1.0000x reference)
import functools

import jax
import jax.numpy as jnp
from jax.experimental import pallas as pl
from jax.experimental.pallas import tpu as pltpu

F32 = jnp.float32
BF16 = jnp.bfloat16
RB = 8192


def _dot(a, b):
    return jnp.dot(a.astype(BF16), b.astype(BF16), preferred_element_type=F32)


def _st(acc, count, g, be):
    m = acc[0] / count
    v = acc[1] / count - m * m
    s = g / jnp.sqrt(v + 1e-5)
    t = be - m * s
    return s[None, :].astype(F32), t[None, :].astype(F32)


def _mm_body(x_ref, s_ref, t_ref, w_ref, b_ref, y_ref, acc_ref, *, pre):
    x = x_ref[...]
    x2 = x.reshape(-1, x.shape[-1])
    if pre:
        x2 = jnp.maximum(x2 * s_ref[...] + t_ref[...], 0.0)
    y2 = _dot(x2, w_ref[...]) + b_ref[...]
    y_ref[...] = y2.reshape(y_ref.shape)

    @pl.when(pl.program_id(0) == 0)
    def _():
        acc_ref[...] = jnp.zeros_like(acc_ref)

    acc_ref[...] += jnp.concatenate(
        [jnp.sum(y2, 0, keepdims=True), jnp.sum(y2 * y2, 0, keepdims=True)], 0)


def _mm_pass(x, s, t, W, b, K, pre=True):
    cin, cout = W.shape
    BN = x.shape[0]
    nb = RB // (K or 1)
    grid = BN // nb
    if K is None:
        xspec = pl.BlockSpec((nb, cin), lambda i: (i, 0))
        yshape = (BN, cout)
        yspec = pl.BlockSpec((nb, cout), lambda i: (i, 0))
    else:
        xspec = pl.BlockSpec((nb, K, cin), lambda i: (i, 0, 0))
        yshape = (BN, K, cout)
        yspec = pl.BlockSpec((nb, K, cout), lambda i: (i, 0, 0))
    vec_in = lambda c: pl.BlockSpec((1, c), lambda i: (0, 0))
    y, acc = pl.pallas_call(
        functools.partial(_mm_body, pre=pre),
        grid=(grid,),
        in_specs=[xspec, vec_in(cin), vec_in(cin),
                  pl.BlockSpec((cin, cout), lambda i: (0, 0)), vec_in(cout)],
        out_specs=[yspec, pl.BlockSpec((2, cout), lambda i: (0, 0))],
        out_shape=[jax.ShapeDtypeStruct(yshape, F32),
                   jax.ShapeDtypeStruct((2, cout), F32)],
    )(x, s, t, W, b)
    return y, acc


def _feat_body(g_ref, wxyz_ref, cmain_ref, cenc_ref,
               wm_ref, wem_ref, we_ref, wee_ref,
               ymain_ref, yenc_ref, accm_ref, acce_ref):
    nb, K, _ = g_ref.shape
    g = g_ref[...]
    gx = g[:, :, 0:3]
    wx = wxyz_ref[...]
    diff = gx - wx[:, None, :]
    euc = jnp.sqrt(jnp.sum(diff * diff, -1, keepdims=True) + 1e-20)
    g2 = g.reshape(nb * K, 128)
    eu2 = euc.reshape(nb * K, 1)
    ym2 = _dot(g2, wm_ref[...]) + eu2 * wem_ref[...]
    ye2 = _dot(g2, we_ref[...]) + eu2 * wee_ref[...]
    ym = ym2.reshape(nb, K, -1) + cmain_ref[...][:, None, :]
    ye = ye2.reshape(nb, K, -1) + cenc_ref[...][:, None, :]
    ymain_ref[...] = ym
    yenc_ref[...] = ye

    @pl.when(pl.program_id(0) == 0)
    def _():
        accm_ref[...] = jnp.zeros_like(accm_ref)
        acce_ref[...] = jnp.zeros_like(acce_ref)

    ym2f = ym.reshape(nb * K, -1)
    ye2f = ye.reshape(nb * K, -1)
    accm_ref[...] += jnp.concatenate(
        [jnp.sum(ym2f, 0, keepdims=True), jnp.sum(ym2f * ym2f, 0, keepdims=True)], 0)
    acce_ref[...] += jnp.concatenate(
        [jnp.sum(ye2f, 0, keepdims=True), jnp.sum(ye2f * ye2f, 0, keepdims=True)], 0)


def _feat_pass(g, wxyz, cmain, cenc, Wm, Wem, We, Wee, K):
    BN = g.shape[0]
    cm = Wm.shape[1]
    ce = We.shape[1]
    nb = RB // K
    grid = BN // nb
    b3 = lambda c: pl.BlockSpec((nb, K, c), lambda i: (i, 0, 0))
    b2 = lambda c: pl.BlockSpec((nb, c), lambda i: (i, 0))
    wfull = lambda a, b_: pl.BlockSpec((a, b_), lambda i: (0, 0))
    return pl.pallas_call(
        _feat_body,
        grid=(grid,),
        in_specs=[b3(128), b2(3), b2(cm), b2(ce),
                  wfull(128, cm), wfull(1, cm),
                  wfull(128, ce), wfull(1, ce)],
        out_specs=[b3(cm), b3(ce),
                   pl.BlockSpec((2, cm), lambda i: (0, 0)),
                   pl.BlockSpec((2, ce), lambda i: (0, 0))],
        out_shape=[jax.ShapeDtypeStruct((BN, K, cm), F32),
                   jax.ShapeDtypeStruct((BN, K, ce), F32),
                   jax.ShapeDtypeStruct((2, cm), F32),
                   jax.ShapeDtypeStruct((2, ce), F32)],
    )(g, wxyz, cmain, cenc, Wm, Wem, We, Wee)


def _comb2_body(ya_ref, sa_ref, ta_ref, yb_ref, sb_ref, tb_ref,
                wa_ref, wb_ref, b_ref, y_ref, feat_ref, acc_ref):
    nb, K, _ = ya_ref.shape
    xa = jnp.maximum(ya_ref[...] * sa_ref[...] + ta_ref[...], 0.0)
    xb = jnp.maximum(yb_ref[...] * sb_ref[...] + tb_ref[...], 0.0)
    feat_ref[...] = xb
    xa2 = xa.reshape(nb * K, -1)
    xb2 = xb.reshape(nb * K, -1)
    y2 = _dot(xa2, wa_ref[...]) + _dot(xb2, wb_ref[...]) + b_ref[...]
    y_ref[...] = y2.reshape(y_ref.shape)

    @pl.when(pl.program_id(0) == 0)
    def _():
        acc_ref[...] = jnp.zeros_like(acc_ref)

    acc_ref[...] += jnp.concatenate(
        [jnp.sum(y2, 0, keepdims=True), jnp.sum(y2 * y2, 0, keepdims=True)], 0)


def _comb2_pass(ya, sa, ta, yb, sb, tb, Wa, Wb, b, K):
    BN = ya.shape[0]
    ca, cout = Wa.shape
    cb = Wb.shape[0]
    nb = RB // K
    grid = BN // nb
    b3 = lambda c: pl.BlockSpec((nb, K, c), lambda i: (i, 0, 0))
    v = lambda c: pl.BlockSpec((1, c), lambda i: (0, 0))
    w = lambda a, b_: pl.BlockSpec((a, b_), lambda i: (0, 0))
    return pl.pallas_call(
        _comb2_body,
        grid=(grid,),
        in_specs=[b3(ca), v(ca), v(ca), b3(cb), v(cb), v(cb),
                  w(ca, cout), w(cb, cout), v(cout)],
        out_specs=[b3(cout), b3(cb), pl.BlockSpec((2, cout), lambda i: (0, 0))],
        out_shape=[jax.ShapeDtypeStruct((BN, K, cout), F32),
                   jax.ShapeDtypeStruct((BN, K, cb), F32),
                   jax.ShapeDtypeStruct((2, cout), F32)],
    )(ya, sa, ta, yb, sb, tb, Wa, Wb, b)


def _addres_body(ya_ref, sa_ref, ta_ref, res_ref, wa_ref, y_ref, acc_ref):
    nb, K, _ = ya_ref.shape
    xa = jnp.maximum(ya_ref[...] * sa_ref[...] + ta_ref[...], 0.0)
    xa2 = xa.reshape(nb * K, -1)
    y2 = _dot(xa2, wa_ref[...]) + res_ref[...].reshape(nb * K, -1)
    y_ref[...] = y2.reshape(y_ref.shape)

    @pl.when(pl.program_id(0) == 0)
    def _():
        acc_ref[...] = jnp.zeros_like(acc_ref)

    acc_ref[...] += jnp.concatenate(
        [jnp.sum(y2, 0, keepdims=True), jnp.sum(y2 * y2, 0, keepdims=True)], 0)


def _addres_pass(ya, sa, ta, res, Wa, K):
    BN = ya.shape[0]
    ca, cout = Wa.shape
    nb = RB // K
    grid = BN // nb
    b3 = lambda c: pl.BlockSpec((nb, K, c), lambda i: (i, 0, 0))
    v = lambda c: pl.BlockSpec((1, c), lambda i: (0, 0))
    return pl.pallas_call(
        _addres_body,
        grid=(grid,),
        in_specs=[b3(ca), v(ca), v(ca), b3(cout),
                  pl.BlockSpec((ca, cout), lambda i: (0, 0))],
        out_specs=[b3(cout), pl.BlockSpec((2, cout), lambda i: (0, 0))],
        out_shape=[jax.ShapeDtypeStruct((BN, K, cout), F32),
                   jax.ShapeDtypeStruct((2, cout), F32)],
    )(ya, sa, ta, res, Wa)


def _smr_body(y_ref, s_ref, t_ref, v_ref, o_ref, *, vlo):
    c = y_ref.shape[-1]
    x = jnp.maximum(y_ref[...] * s_ref[...] + t_ref[...], 0.0)
    m = jnp.max(x, axis=1, keepdims=True)
    e = jnp.exp(x - m)
    w = e / jnp.sum(e, axis=1, keepdims=True)
    o_ref[...] = jnp.sum(w * v_ref[...][:, :, vlo:vlo + c], axis=1)


def _smr_pass(y, s, t, vals, K, vlo=0):
    BN, _, c = y.shape
    cv = vals.shape[-1]
    nb = RB // K
    grid = BN // nb
    b3 = lambda cc: pl.BlockSpec((nb, K, cc), lambda i: (i, 0, 0))
    v = lambda cc: pl.BlockSpec((1, cc), lambda i: (0, 0))
    return pl.pallas_call(
        functools.partial(_smr_body, vlo=vlo),
        grid=(grid,),
        in_specs=[b3(c), v(c), v(c), b3(cv)],
        out_specs=pl.BlockSpec((nb, c), lambda i: (i, 0)),
        out_shape=jax.ShapeDtypeStruct((BN, c), F32),
    )(y, s, t, vals)


QB = 1024

_INF = 3.0e38


def _knn_body(q_ref, kt_ref, out_ref, d_ref, *, k, n):
    q = q_ref[0]
    kt = kt_ref[0]
    qb = q.astype(jnp.bfloat16).astype(F32)
    ktb = kt.astype(jnp.bfloat16).astype(F32)
    dot = (qb[:, 0:1] * ktb[0:1, :] + qb[:, 1:2] * ktb[1:2, :]
           + qb[:, 2:3] * ktb[2:3, :])
    q2 = jnp.sum(q * q, axis=1, keepdims=True)
    k2 = jnp.sum(kt * kt, axis=0, keepdims=True)
    d_ref[...] = (-2.0 * dot + q2) + k2
    iota_n = jax.lax.broadcasted_iota(jnp.int32, (QB, n), 1)
    iota_k = jax.lax.broadcasted_iota(jnp.int32, (QB, k), 1)

    def body(j, carry):
        am_prev, acc = carry
        dc = jnp.where(iota_n == am_prev, _INF, d_ref[...])
        d_ref[...] = dc
        m = jnp.min(dc, axis=1, keepdims=True)
        am = jnp.min(jnp.where(dc <= m, iota_n, n), axis=1, keepdims=True)
        return am, jnp.where(iota_k == j, am, acc)

    _, res = jax.lax.fori_loop(
        0, k, body,
        (jnp.full((QB, 1), -1, jnp.int32), jnp.zeros((QB, k), jnp.int32)))
    out_ref[0] = res


def _knn(keys, queries, k):
    B, N, _ = keys.shape
    kt = keys.transpose(0, 2, 1)
    return pl.pallas_call(
        functools.partial(_knn_body, k=k, n=N),
        grid=(B, N // QB),
        in_specs=[pl.BlockSpec((1, QB, 3), lambda b, i: (b, i, 0)),
                  pl.BlockSpec((1, 3, N), lambda b, i: (b, 0, 0))],
        out_specs=pl.BlockSpec((1, QB, k), lambda b, i: (b, i, 0)),
        out_shape=jax.ShapeDtypeStruct((B, N, k), jnp.int32),
        scratch_shapes=[pltpu.VMEM((QB, N), F32)],
    )(queries, kt)


def _group(p, idx):
    return jax.vmap(lambda pp, ii: pp[ii])(p, idx)


_CH = 128


def _sc_gather(table, idx_flat):
    from jax.experimental.pallas import tpu_sc as plsc

    M = idx_flat.shape[0]
    NW = 32
    per_w = M // NW
    nch = per_w // _CH
    idx2d = idx_flat.reshape(M // _CH, _CH)
    mesh = plsc.VectorSubcoreMesh(core_axis_name="c", subcore_axis_name="s")

    @functools.partial(
        pl.kernel, mesh=mesh,
        out_type=jax.ShapeDtypeStruct((M, 128), F32),
        scratch_types=[pltpu.VMEM((nch, _CH), jnp.int32),
                       pltpu.VMEM((_CH, 128), F32),
                       pltpu.SemaphoreType.DMA],
    )
    def k(tab_hbm, idx_hbm, out_hbm, idx_v, buf0, sem0):
        wid = jax.lax.axis_index("s") * 2 + jax.lax.axis_index("c")
        base = wid * per_w
        pltpu.sync_copy(idx_hbm.at[pl.ds(wid * nch, nch)], idx_v)

        def body(j, carry):
            pltpu.async_copy(tab_hbm.at[idx_v.at[j]], buf0, sem0).wait()
            pltpu.sync_copy(buf0, out_hbm.at[pl.ds(base + j * _CH, _CH)])
            return carry

        jax.lax.fori_loop(0, nch, body, 0)

    return k(table, idx2d)


def kernel(warped_xyz, warped_points, f2_xyz, f2_points, params):
    B, N, C = warped_points.shape
    BN = B * N
    Kq, Ks = 32, 16
    M1 = float(BN * Kq)
    M2 = float(BN * Ks)

    (W1, b1, g1, be1), (W2, b2, g2, be2), (W3, b3, g3, be3) = params['mlp1']
    We, be_, ge, bee = params['pi_enc']
    (W4, b4, g4, be4), (W5, b5, g5, be5) = params['mlp2']
    We2, be2_, ge2, bee2 = params['pc_enc']
    (W8, b8, g8, be8), (W9, b9, g9, be9) = params['mlp2_2']

    A_wx, A_gx, A_df = W1[:, 0:3], W1[:, 3:6], W1[:, 6:9]
    A_eu, A_wp, A_gp = W1[:, 9:10], W1[:, 10:10 + C], W1[:, 10 + C:10 + 2 * C]
    E_wx, E_gx, E_df, E_eu = We[:, 0:3], We[:, 3:6], We[:, 6:9], We[:, 9:10]
    E2_wx, E2_gx, E2_df, E2_eu = We2[:, 0:3], We2[:, 3:6], We2[:, 6:9], We2[:, 9:10]
    W8a, W8m, W8g = W8[:, 0:C], W8[:, C:2 * C], W8[:, 2 * C:3 * C]

    zc3 = jnp.zeros((3, 128), F32)
    z64_3 = jnp.zeros((C, 64), F32)
    Wc = jnp.concatenate([
        jnp.concatenate([(A_wx - A_df).T, A_wp.T], 0),
        jnp.concatenate([(E_wx - E_df).T, z64_3], 0),
        jnp.concatenate([(E2_wx - E2_df).T, z64_3], 0),
        jnp.concatenate([zc3, W8m.T], 0),
    ], 1)
    bc = jnp.concatenate([b1, be_, be2_, b8])[None, :]
    cw = jnp.concatenate([warped_xyz, warped_points], -1).reshape(BN, 3 + C)
    one67 = jnp.ones((1, 3 + C), F32)
    zero67 = jnp.zeros((1, 3 + C), F32)
    c_all, _ = _mm_pass(cw, one67, zero67, Wc, bc, None, pre=False)
    c1, c1e = c_all[:, 0:128], c_all[:, 128:192]
    c7e, c7a = c_all[:, 192:256], c_all[:, 256:384]

    off = (jnp.arange(B, dtype=jnp.int32) * N)[:, None, None]
    pad13 = jnp.zeros((B, N, 13), F32)
    pad48 = jnp.zeros((B, N, 48), F32)
    idx1 = _knn(f2_xyz, warped_xyz, Kq)
    tab1 = jnp.concatenate([f2_xyz, pad13, f2_points, pad48], -1).reshape(BN, 128)
    gg1 = _sc_gather(tab1, (idx1 + off).reshape(BN * Kq)).reshape(BN, Kq, 128)
    idx2 = _knn(warped_xyz, warped_xyz, Ks)
    wxyz2 = warped_xyz.reshape(BN, 3)

    Wm1 = jnp.zeros((128, 128), F32).at[0:3].set((A_gx + A_df).T) \
        .at[16:16 + C].set(A_gp.T)
    We1 = jnp.zeros((128, 64), F32).at[0:3].set((E_gx + E_df).T)
    y1, yenc, acc1, acce = _feat_pass(gg1, wxyz2, c1, c1e,
                                      Wm1, A_eu.T, We1, E_eu.T, Kq)

    s1, t1 = _st(acc1, M1, g1, be1)
    y2, acc2 = _mm_pass(y1, s1, t1, W2.T, b2[None], Kq)
    s2, t2 = _st(acc2, M1, g2, be2)
    y3, acc3 = _mm_pass(y2, s2, t2, W3.T, b3[None], Kq)
    s3, t3 = _st(acc3, M1, g3, be3)
    se, te = _st(acce, M1, ge, bee)
    y4, feat, acc4 = _comb2_pass(yenc, se, te, y3, s3, t3,
                                 W4[:, 0:C].T, W4[:, C:2 * C].T, b4[None], Kq)
    s4, t4 = _st(acc4, M1, g4, be4)
    y5, acc5 = _mm_pass(y4, s4, t4, W5.T, b5[None], Kq)
    s5, t5 = _st(acc5, M1, g5, be5)
    out1 = _smr_pass(y5, s5, t5, feat, Kq)

    tab2 = jnp.concatenate(
        [warped_xyz.reshape(BN, 3), pad13.reshape(BN, 13), out1,
         pad48.reshape(BN, 48)], -1)
    gg2 = _sc_gather(tab2, (idx2 + off).reshape(BN * Ks)).reshape(BN, Ks, 128)

    z1_128 = jnp.zeros((1, 128), F32)
    Wm2 = jnp.zeros((128, 128), F32).at[16:16 + C].set(W8g.T)
    We2p = jnp.zeros((128, 64), F32).at[0:3].set((E2_gx + E2_df).T)
    ya, yenc2, _, acce2 = _feat_pass(gg2, wxyz2, c7a, c7e,
                                     Wm2, z1_128, We2p, E2_eu.T, Ks)

    se2, te2 = _st(acce2, M2, ge2, bee2)
    y8, acc8 = _addres_pass(yenc2, se2, te2, ya, W8a.T, Ks)
    s8, t8 = _st(acc8, M2, g8, be8)
    y9, acc9 = _mm_pass(y8, s8, t8, W9.T, b9[None], Ks)
    s9, t9 = _st(acc9, M2, g9, be9)
    out = _smr_pass(y9, s9, t9, gg2, Ks, vlo=16)
    return out.reshape(B, N, C)


# speedup vs baseline: 1.0273x; 1.0273x over previous
import functools

import jax
import jax.numpy as jnp
from jax.experimental import pallas as pl
from jax.experimental.pallas import tpu as pltpu

F32 = jnp.float32
BF16 = jnp.bfloat16
RB = 8192


def _dot(a, b):
    return jnp.dot(a.astype(BF16), b.astype(BF16), preferred_element_type=F32)


def _st(acc, count, g, be):
    m = acc[0] / count
    v = acc[1] / count - m * m
    s = g / jnp.sqrt(v + 1e-5)
    t = be - m * s
    return s[None, :].astype(F32), t[None, :].astype(F32)


def _mm_body(x_ref, s_ref, t_ref, w_ref, b_ref, y_ref, acc_ref, *, pre):
    x = x_ref[...]
    x2 = x.reshape(-1, x.shape[-1])
    if pre:
        x2 = jnp.maximum(x2 * s_ref[...] + t_ref[...], 0.0)
    y2 = _dot(x2, w_ref[...]) + b_ref[...]
    y_ref[...] = y2.reshape(y_ref.shape)

    @pl.when(pl.program_id(0) == 0)
    def _():
        acc_ref[...] = jnp.zeros_like(acc_ref)

    acc_ref[...] += jnp.concatenate(
        [jnp.sum(y2, 0, keepdims=True), jnp.sum(y2 * y2, 0, keepdims=True)], 0)


def _mm_pass(x, s, t, W, b, K, pre=True):
    cin, cout = W.shape
    BN = x.shape[0]
    nb = RB // (K or 1)
    grid = BN // nb
    if K is None:
        xspec = pl.BlockSpec((nb, cin), lambda i: (i, 0))
        yshape = (BN, cout)
        yspec = pl.BlockSpec((nb, cout), lambda i: (i, 0))
    else:
        xspec = pl.BlockSpec((nb, K, cin), lambda i: (i, 0, 0))
        yshape = (BN, K, cout)
        yspec = pl.BlockSpec((nb, K, cout), lambda i: (i, 0, 0))
    vec_in = lambda c: pl.BlockSpec((1, c), lambda i: (0, 0))
    y, acc = pl.pallas_call(
        functools.partial(_mm_body, pre=pre),
        grid=(grid,),
        in_specs=[xspec, vec_in(cin), vec_in(cin),
                  pl.BlockSpec((cin, cout), lambda i: (0, 0)), vec_in(cout)],
        out_specs=[yspec, pl.BlockSpec((2, cout), lambda i: (0, 0))],
        out_shape=[jax.ShapeDtypeStruct(yshape, F32),
                   jax.ShapeDtypeStruct((2, cout), F32)],
    )(x, s, t, W, b)
    return y, acc


def _feat_body(g_ref, wxyz_ref, cmain_ref, cenc_ref,
               wm_ref, wem_ref, we_ref, wee_ref,
               ymain_ref, yenc_ref, accm_ref, acce_ref):
    nb, K, _ = g_ref.shape
    g = g_ref[...]
    gx = g[:, :, 0:3]
    wx = wxyz_ref[...]
    diff = gx - wx[:, None, :]
    euc = jnp.sqrt(jnp.sum(diff * diff, -1, keepdims=True) + 1e-20)
    g2 = g.reshape(nb * K, 128)
    eu2 = euc.reshape(nb * K, 1)
    ym2 = _dot(g2, wm_ref[...]) + eu2 * wem_ref[...]
    ye2 = _dot(g2, we_ref[...]) + eu2 * wee_ref[...]
    ym = ym2.reshape(nb, K, -1) + cmain_ref[...][:, None, :]
    ye = ye2.reshape(nb, K, -1) + cenc_ref[...][:, None, :]
    ymain_ref[...] = ym
    yenc_ref[...] = ye

    @pl.when(pl.program_id(0) == 0)
    def _():
        accm_ref[...] = jnp.zeros_like(accm_ref)
        acce_ref[...] = jnp.zeros_like(acce_ref)

    ym2f = ym.reshape(nb * K, -1)
    ye2f = ye.reshape(nb * K, -1)
    accm_ref[...] += jnp.concatenate(
        [jnp.sum(ym2f, 0, keepdims=True), jnp.sum(ym2f * ym2f, 0, keepdims=True)], 0)
    acce_ref[...] += jnp.concatenate(
        [jnp.sum(ye2f, 0, keepdims=True), jnp.sum(ye2f * ye2f, 0, keepdims=True)], 0)


def _feat_pass(g, wxyz, cmain, cenc, Wm, Wem, We, Wee, K):
    BN = g.shape[0]
    cm = Wm.shape[1]
    ce = We.shape[1]
    nb = RB // K
    grid = BN // nb
    b3 = lambda c: pl.BlockSpec((nb, K, c), lambda i: (i, 0, 0))
    b2 = lambda c: pl.BlockSpec((nb, c), lambda i: (i, 0))
    wfull = lambda a, b_: pl.BlockSpec((a, b_), lambda i: (0, 0))
    return pl.pallas_call(
        _feat_body,
        grid=(grid,),
        in_specs=[b3(128), b2(3), b2(cm), b2(ce),
                  wfull(128, cm), wfull(1, cm),
                  wfull(128, ce), wfull(1, ce)],
        out_specs=[b3(cm), b3(ce),
                   pl.BlockSpec((2, cm), lambda i: (0, 0)),
                   pl.BlockSpec((2, ce), lambda i: (0, 0))],
        out_shape=[jax.ShapeDtypeStruct((BN, K, cm), F32),
                   jax.ShapeDtypeStruct((BN, K, ce), F32),
                   jax.ShapeDtypeStruct((2, cm), F32),
                   jax.ShapeDtypeStruct((2, ce), F32)],
    )(g, wxyz, cmain, cenc, Wm, Wem, We, Wee)


def _comb2_body(ya_ref, sa_ref, ta_ref, yb_ref, sb_ref, tb_ref,
                wa_ref, wb_ref, b_ref, y_ref, feat_ref, acc_ref):
    nb, K, _ = ya_ref.shape
    xa = jnp.maximum(ya_ref[...] * sa_ref[...] + ta_ref[...], 0.0)
    xb = jnp.maximum(yb_ref[...] * sb_ref[...] + tb_ref[...], 0.0)
    feat_ref[...] = xb
    xa2 = xa.reshape(nb * K, -1)
    xb2 = xb.reshape(nb * K, -1)
    y2 = _dot(xa2, wa_ref[...]) + _dot(xb2, wb_ref[...]) + b_ref[...]
    y_ref[...] = y2.reshape(y_ref.shape)

    @pl.when(pl.program_id(0) == 0)
    def _():
        acc_ref[...] = jnp.zeros_like(acc_ref)

    acc_ref[...] += jnp.concatenate(
        [jnp.sum(y2, 0, keepdims=True), jnp.sum(y2 * y2, 0, keepdims=True)], 0)


def _comb2_pass(ya, sa, ta, yb, sb, tb, Wa, Wb, b, K):
    BN = ya.shape[0]
    ca, cout = Wa.shape
    cb = Wb.shape[0]
    nb = RB // K
    grid = BN // nb
    b3 = lambda c: pl.BlockSpec((nb, K, c), lambda i: (i, 0, 0))
    v = lambda c: pl.BlockSpec((1, c), lambda i: (0, 0))
    w = lambda a, b_: pl.BlockSpec((a, b_), lambda i: (0, 0))
    return pl.pallas_call(
        _comb2_body,
        grid=(grid,),
        in_specs=[b3(ca), v(ca), v(ca), b3(cb), v(cb), v(cb),
                  w(ca, cout), w(cb, cout), v(cout)],
        out_specs=[b3(cout), b3(cb), pl.BlockSpec((2, cout), lambda i: (0, 0))],
        out_shape=[jax.ShapeDtypeStruct((BN, K, cout), F32),
                   jax.ShapeDtypeStruct((BN, K, cb), F32),
                   jax.ShapeDtypeStruct((2, cout), F32)],
    )(ya, sa, ta, yb, sb, tb, Wa, Wb, b)


def _addres_body(ya_ref, sa_ref, ta_ref, res_ref, wa_ref, y_ref, acc_ref):
    nb, K, _ = ya_ref.shape
    xa = jnp.maximum(ya_ref[...] * sa_ref[...] + ta_ref[...], 0.0)
    xa2 = xa.reshape(nb * K, -1)
    y2 = _dot(xa2, wa_ref[...]) + res_ref[...].reshape(nb * K, -1)
    y_ref[...] = y2.reshape(y_ref.shape)

    @pl.when(pl.program_id(0) == 0)
    def _():
        acc_ref[...] = jnp.zeros_like(acc_ref)

    acc_ref[...] += jnp.concatenate(
        [jnp.sum(y2, 0, keepdims=True), jnp.sum(y2 * y2, 0, keepdims=True)], 0)


def _addres_pass(ya, sa, ta, res, Wa, K):
    BN = ya.shape[0]
    ca, cout = Wa.shape
    nb = RB // K
    grid = BN // nb
    b3 = lambda c: pl.BlockSpec((nb, K, c), lambda i: (i, 0, 0))
    v = lambda c: pl.BlockSpec((1, c), lambda i: (0, 0))
    return pl.pallas_call(
        _addres_body,
        grid=(grid,),
        in_specs=[b3(ca), v(ca), v(ca), b3(cout),
                  pl.BlockSpec((ca, cout), lambda i: (0, 0))],
        out_specs=[b3(cout), pl.BlockSpec((2, cout), lambda i: (0, 0))],
        out_shape=[jax.ShapeDtypeStruct((BN, K, cout), F32),
                   jax.ShapeDtypeStruct((2, cout), F32)],
    )(ya, sa, ta, res, Wa)


def _smr_body(y_ref, s_ref, t_ref, v_ref, o_ref, *, vlo):
    c = y_ref.shape[-1]
    x = jnp.maximum(y_ref[...] * s_ref[...] + t_ref[...], 0.0)
    m = jnp.max(x, axis=1, keepdims=True)
    e = jnp.exp(x - m)
    w = e / jnp.sum(e, axis=1, keepdims=True)
    o_ref[...] = jnp.sum(w * v_ref[...][:, :, vlo:vlo + c], axis=1)


def _smr_pass(y, s, t, vals, K, vlo=0):
    BN, _, c = y.shape
    cv = vals.shape[-1]
    nb = RB // K
    grid = BN // nb
    b3 = lambda cc: pl.BlockSpec((nb, K, cc), lambda i: (i, 0, 0))
    v = lambda cc: pl.BlockSpec((1, cc), lambda i: (0, 0))
    return pl.pallas_call(
        functools.partial(_smr_body, vlo=vlo),
        grid=(grid,),
        in_specs=[b3(c), v(c), v(c), b3(cv)],
        out_specs=pl.BlockSpec((nb, c), lambda i: (i, 0)),
        out_shape=jax.ShapeDtypeStruct((BN, c), F32),
    )(y, s, t, vals)


QB = 1024

_INF = 3.0e38


def _knn_body(q_ref, kt_ref, out_ref, d_ref, *, k, n):
    q = q_ref[0]
    kt = kt_ref[0]
    qb = q.astype(jnp.bfloat16).astype(F32)
    ktb = kt.astype(jnp.bfloat16).astype(F32)
    dot = (qb[:, 0:1] * ktb[0:1, :] + qb[:, 1:2] * ktb[1:2, :]
           + qb[:, 2:3] * ktb[2:3, :])
    q2 = jnp.sum(q * q, axis=1, keepdims=True)
    k2 = jnp.sum(kt * kt, axis=0, keepdims=True)
    d_ref[...] = (-2.0 * dot + q2) + k2
    iota_n = jax.lax.broadcasted_iota(jnp.int32, (QB, n), 1)
    iota_k = jax.lax.broadcasted_iota(jnp.int32, (QB, k), 1)

    def body(j, acc):
        dc = d_ref[...]
        m = jnp.min(dc, axis=1, keepdims=True)
        am = jnp.min(jnp.where(dc <= m, iota_n, n), axis=1, keepdims=True)
        d_ref[...] = jnp.where(iota_n == am, _INF, dc)
        return jnp.where(iota_k == j, am, acc)

    out_ref[0] = jax.lax.fori_loop(0, k, body, jnp.zeros((QB, k), jnp.int32))


def _knn(keys, queries, k):
    B, N, _ = keys.shape
    kt = keys.transpose(0, 2, 1)
    return pl.pallas_call(
        functools.partial(_knn_body, k=k, n=N),
        grid=(B, N // QB),
        in_specs=[pl.BlockSpec((1, QB, 3), lambda b, i: (b, i, 0)),
                  pl.BlockSpec((1, 3, N), lambda b, i: (b, 0, 0))],
        out_specs=pl.BlockSpec((1, QB, k), lambda b, i: (b, i, 0)),
        out_shape=jax.ShapeDtypeStruct((B, N, k), jnp.int32),
        scratch_shapes=[pltpu.VMEM((QB, N), F32)],
    )(queries, kt)


def _group(p, idx):
    return jax.vmap(lambda pp, ii: pp[ii])(p, idx)


_CH = 128


def _sc_gather(table, idx_flat):
    from jax.experimental.pallas import tpu_sc as plsc

    M = idx_flat.shape[0]
    NW = 32
    per_w = M // NW
    nch = per_w // _CH
    idx2d = idx_flat.reshape(M // _CH, _CH)
    mesh = plsc.VectorSubcoreMesh(core_axis_name="c", subcore_axis_name="s")

    @functools.partial(
        pl.kernel, mesh=mesh,
        out_type=jax.ShapeDtypeStruct((M, 128), F32),
        scratch_types=[pltpu.VMEM((nch, _CH), jnp.int32),
                       pltpu.VMEM((_CH, 128), F32),
                       pltpu.VMEM((_CH, 128), F32),
                       pltpu.SemaphoreType.DMA,
                       pltpu.SemaphoreType.DMA],
    )
    def k(tab_hbm, idx_hbm, out_hbm, idx_v, buf0, buf1, sem0, sem1):
        wid = jax.lax.axis_index("s") * 2 + jax.lax.axis_index("c")
        base = wid * per_w
        pltpu.sync_copy(idx_hbm.at[pl.ds(wid * nch, nch)], idx_v)
        pltpu.async_copy(tab_hbm.at[idx_v.at[0]], buf0, sem0)

        def body(i, carry):
            j0 = 2 * i
            pltpu.async_copy(tab_hbm.at[idx_v.at[j0 + 1]], buf1, sem1)
            pltpu.make_async_copy(tab_hbm.at[idx_v.at[j0]], buf0, sem0).wait()
            pltpu.sync_copy(buf0, out_hbm.at[pl.ds(base + j0 * _CH, _CH)])

            @pl.when(i + 1 < nch // 2)
            def _():
                pltpu.async_copy(tab_hbm.at[idx_v.at[j0 + 2]], buf0, sem0)

            pltpu.make_async_copy(tab_hbm.at[idx_v.at[j0 + 1]], buf1, sem1).wait()
            pltpu.sync_copy(buf1, out_hbm.at[pl.ds(base + (j0 + 1) * _CH, _CH)])
            return carry

        jax.lax.fori_loop(0, nch // 2, body, 0)

    return k(table, idx2d)


def kernel(warped_xyz, warped_points, f2_xyz, f2_points, params):
    B, N, C = warped_points.shape
    BN = B * N
    Kq, Ks = 32, 16
    M1 = float(BN * Kq)
    M2 = float(BN * Ks)

    (W1, b1, g1, be1), (W2, b2, g2, be2), (W3, b3, g3, be3) = params['mlp1']
    We, be_, ge, bee = params['pi_enc']
    (W4, b4, g4, be4), (W5, b5, g5, be5) = params['mlp2']
    We2, be2_, ge2, bee2 = params['pc_enc']
    (W8, b8, g8, be8), (W9, b9, g9, be9) = params['mlp2_2']

    A_wx, A_gx, A_df = W1[:, 0:3], W1[:, 3:6], W1[:, 6:9]
    A_eu, A_wp, A_gp = W1[:, 9:10], W1[:, 10:10 + C], W1[:, 10 + C:10 + 2 * C]
    E_wx, E_gx, E_df, E_eu = We[:, 0:3], We[:, 3:6], We[:, 6:9], We[:, 9:10]
    E2_wx, E2_gx, E2_df, E2_eu = We2[:, 0:3], We2[:, 3:6], We2[:, 6:9], We2[:, 9:10]
    W8a, W8m, W8g = W8[:, 0:C], W8[:, C:2 * C], W8[:, 2 * C:3 * C]

    zc3 = jnp.zeros((3, 128), F32)
    z64_3 = jnp.zeros((C, 64), F32)
    Wc = jnp.concatenate([
        jnp.concatenate([(A_wx - A_df).T, A_wp.T], 0),
        jnp.concatenate([(E_wx - E_df).T, z64_3], 0),
        jnp.concatenate([(E2_wx - E2_df).T, z64_3], 0),
        jnp.concatenate([zc3, W8m.T], 0),
    ], 1)
    bc = jnp.concatenate([b1, be_, be2_, b8])[None, :]
    cw = jnp.concatenate([warped_xyz, warped_points], -1).reshape(BN, 3 + C)
    one67 = jnp.ones((1, 3 + C), F32)
    zero67 = jnp.zeros((1, 3 + C), F32)
    c_all, _ = _mm_pass(cw, one67, zero67, Wc, bc, None, pre=False)
    c1, c1e = c_all[:, 0:128], c_all[:, 128:192]
    c7e, c7a = c_all[:, 192:256], c_all[:, 256:384]

    off = (jnp.arange(B, dtype=jnp.int32) * N)[:, None, None]
    pad13 = jnp.zeros((B, N, 13), F32)
    pad48 = jnp.zeros((B, N, 48), F32)
    idx1 = _knn(f2_xyz, warped_xyz, Kq)
    tab1 = jnp.concatenate([f2_xyz, pad13, f2_points, pad48], -1).reshape(BN, 128)
    gg1 = _sc_gather(tab1, (idx1 + off).reshape(BN * Kq)).reshape(BN, Kq, 128)
    idx2 = _knn(warped_xyz, warped_xyz, Ks)
    wxyz2 = warped_xyz.reshape(BN, 3)

    Wm1 = jnp.zeros((128, 128), F32).at[0:3].set((A_gx + A_df).T) \
        .at[16:16 + C].set(A_gp.T)
    We1 = jnp.zeros((128, 64), F32).at[0:3].set((E_gx + E_df).T)
    y1, yenc, acc1, acce = _feat_pass(gg1, wxyz2, c1, c1e,
                                      Wm1, A_eu.T, We1, E_eu.T, Kq)

    s1, t1 = _st(acc1, M1, g1, be1)
    y2, acc2 = _mm_pass(y1, s1, t1, W2.T, b2[None], Kq)
    s2, t2 = _st(acc2, M1, g2, be2)
    y3, acc3 = _mm_pass(y2, s2, t2, W3.T, b3[None], Kq)
    s3, t3 = _st(acc3, M1, g3, be3)
    se, te = _st(acce, M1, ge, bee)
    y4, feat, acc4 = _comb2_pass(yenc, se, te, y3, s3, t3,
                                 W4[:, 0:C].T, W4[:, C:2 * C].T, b4[None], Kq)
    s4, t4 = _st(acc4, M1, g4, be4)
    y5, acc5 = _mm_pass(y4, s4, t4, W5.T, b5[None], Kq)
    s5, t5 = _st(acc5, M1, g5, be5)
    out1 = _smr_pass(y5, s5, t5, feat, Kq)

    tab2 = jnp.concatenate(
        [warped_xyz.reshape(BN, 3), pad13.reshape(BN, 13), out1,
         pad48.reshape(BN, 48)], -1)
    gg2 = _sc_gather(tab2, (idx2 + off).reshape(BN * Ks)).reshape(BN, Ks, 128)

    z1_128 = jnp.zeros((1, 128), F32)
    Wm2 = jnp.zeros((128, 128), F32).at[16:16 + C].set(W8g.T)
    We2p = jnp.zeros((128, 64), F32).at[0:3].set((E2_gx + E2_df).T)
    ya, yenc2, _, acce2 = _feat_pass(gg2, wxyz2, c7a, c7e,
                                     Wm2, z1_128, We2p, E2_eu.T, Ks)

    se2, te2 = _st(acce2, M2, ge2, bee2)
    y8, acc8 = _addres_pass(yenc2, se2, te2, ya, W8a.T, Ks)
    s8, t8 = _st(acc8, M2, g8, be8)
    y9, acc9 = _mm_pass(y8, s8, t8, W9.T, b9[None], Ks)
    s9, t9 = _st(acc9, M2, g9, be9)
    out = _smr_pass(y9, s9, t9, gg2, Ks, vlo=16)
    return out.reshape(B, N, C)


# speedup vs baseline: 1.0970x; 1.0679x over previous
import functools

import jax
import jax.numpy as jnp
from jax.experimental import pallas as pl
from jax.experimental.pallas import tpu as pltpu

F32 = jnp.float32
BF16 = jnp.bfloat16
RB = 8192


def _dot(a, b):
    return jnp.dot(a.astype(BF16), b.astype(BF16), preferred_element_type=F32)


def _st(acc, count, g, be):
    m = acc[0] / count
    v = acc[1] / count - m * m
    s = g / jnp.sqrt(v + 1e-5)
    t = be - m * s
    return s[None, :].astype(F32), t[None, :].astype(F32)


def _mm_body(x_ref, s_ref, t_ref, w_ref, b_ref, y_ref, acc_ref, *, pre):
    x = x_ref[...]
    x2 = x.reshape(-1, x.shape[-1])
    if pre:
        x2 = jnp.maximum(x2 * s_ref[...] + t_ref[...], 0.0)
    y2 = _dot(x2, w_ref[...]) + b_ref[...]
    y_ref[...] = y2.reshape(y_ref.shape).astype(y_ref.dtype)

    @pl.when(pl.program_id(0) == 0)
    def _():
        acc_ref[...] = jnp.zeros_like(acc_ref)

    acc_ref[...] += jnp.concatenate(
        [jnp.sum(y2, 0, keepdims=True), jnp.sum(y2 * y2, 0, keepdims=True)], 0)


def _mm_pass(x, s, t, W, b, K, pre=True, out_dtype=BF16):
    cin, cout = W.shape
    BN = x.shape[0]
    nb = RB // (K or 1)
    grid = BN // nb
    if K is None:
        xspec = pl.BlockSpec((nb, cin), lambda i: (i, 0))
        yshape = (BN, cout)
        yspec = pl.BlockSpec((nb, cout), lambda i: (i, 0))
    else:
        xspec = pl.BlockSpec((nb, K, cin), lambda i: (i, 0, 0))
        yshape = (BN, K, cout)
        yspec = pl.BlockSpec((nb, K, cout), lambda i: (i, 0, 0))
    vec_in = lambda c: pl.BlockSpec((1, c), lambda i: (0, 0))
    y, acc = pl.pallas_call(
        functools.partial(_mm_body, pre=pre),
        grid=(grid,),
        in_specs=[xspec, vec_in(cin), vec_in(cin),
                  pl.BlockSpec((cin, cout), lambda i: (0, 0)), vec_in(cout)],
        out_specs=[yspec, pl.BlockSpec((2, cout), lambda i: (0, 0))],
        out_shape=[jax.ShapeDtypeStruct(yshape, out_dtype),
                   jax.ShapeDtypeStruct((2, cout), F32)],
    )(x, s, t, W, b)
    return y, acc


def _feat_body(g_ref, wxyz_ref, cmain_ref, cenc_ref,
               wm_ref, wem_ref, we_ref, wee_ref,
               ymain_ref, yenc_ref, accm_ref, acce_ref):
    nb, K, _ = g_ref.shape
    g = g_ref[...]
    gx = g[:, :, 0:3]
    wx = wxyz_ref[...]
    diff = gx - wx[:, None, :]
    euc = jnp.sqrt(jnp.sum(diff * diff, -1, keepdims=True) + 1e-20)
    g2 = g.reshape(nb * K, 128)
    eu2 = euc.reshape(nb * K, 1)
    ym2 = _dot(g2, wm_ref[...]) + eu2 * wem_ref[...]
    ye2 = _dot(g2, we_ref[...]) + eu2 * wee_ref[...]
    ym = ym2.reshape(nb, K, -1) + cmain_ref[...][:, None, :]
    ye = ye2.reshape(nb, K, -1) + cenc_ref[...][:, None, :]
    ymain_ref[...] = ym.astype(ymain_ref.dtype)
    yenc_ref[...] = ye.astype(yenc_ref.dtype)

    @pl.when(pl.program_id(0) == 0)
    def _():
        accm_ref[...] = jnp.zeros_like(accm_ref)
        acce_ref[...] = jnp.zeros_like(acce_ref)

    ym2f = ym.reshape(nb * K, -1)
    ye2f = ye.reshape(nb * K, -1)
    accm_ref[...] += jnp.concatenate(
        [jnp.sum(ym2f, 0, keepdims=True), jnp.sum(ym2f * ym2f, 0, keepdims=True)], 0)
    acce_ref[...] += jnp.concatenate(
        [jnp.sum(ye2f, 0, keepdims=True), jnp.sum(ye2f * ye2f, 0, keepdims=True)], 0)


def _feat_pass(g, wxyz, cmain, cenc, Wm, Wem, We, Wee, K):
    BN = g.shape[0]
    cm = Wm.shape[1]
    ce = We.shape[1]
    nb = RB // K
    grid = BN // nb
    b3 = lambda c: pl.BlockSpec((nb, K, c), lambda i: (i, 0, 0))
    b2 = lambda c: pl.BlockSpec((nb, c), lambda i: (i, 0))
    wfull = lambda a, b_: pl.BlockSpec((a, b_), lambda i: (0, 0))
    return pl.pallas_call(
        _feat_body,
        grid=(grid,),
        in_specs=[b3(128), b2(3), b2(cm), b2(ce),
                  wfull(128, cm), wfull(1, cm),
                  wfull(128, ce), wfull(1, ce)],
        out_specs=[b3(cm), b3(ce),
                   pl.BlockSpec((2, cm), lambda i: (0, 0)),
                   pl.BlockSpec((2, ce), lambda i: (0, 0))],
        out_shape=[jax.ShapeDtypeStruct((BN, K, cm), BF16),
                   jax.ShapeDtypeStruct((BN, K, ce), BF16),
                   jax.ShapeDtypeStruct((2, cm), F32),
                   jax.ShapeDtypeStruct((2, ce), F32)],
    )(g, wxyz, cmain, cenc, Wm, Wem, We, Wee)


def _comb2_body(ya_ref, sa_ref, ta_ref, yb_ref, sb_ref, tb_ref,
                wa_ref, wb_ref, b_ref, y_ref, feat_ref, acc_ref):
    nb, K, _ = ya_ref.shape
    xa = jnp.maximum(ya_ref[...] * sa_ref[...] + ta_ref[...], 0.0)
    xb = jnp.maximum(yb_ref[...] * sb_ref[...] + tb_ref[...], 0.0)
    feat_ref[...] = xb.astype(feat_ref.dtype)
    xa2 = xa.reshape(nb * K, -1)
    xb2 = xb.reshape(nb * K, -1)
    y2 = _dot(xa2, wa_ref[...]) + _dot(xb2, wb_ref[...]) + b_ref[...]
    y_ref[...] = y2.reshape(y_ref.shape).astype(y_ref.dtype)

    @pl.when(pl.program_id(0) == 0)
    def _():
        acc_ref[...] = jnp.zeros_like(acc_ref)

    acc_ref[...] += jnp.concatenate(
        [jnp.sum(y2, 0, keepdims=True), jnp.sum(y2 * y2, 0, keepdims=True)], 0)


def _comb2_pass(ya, sa, ta, yb, sb, tb, Wa, Wb, b, K):
    BN = ya.shape[0]
    ca, cout = Wa.shape
    cb = Wb.shape[0]
    nb = RB // K
    grid = BN // nb
    b3 = lambda c: pl.BlockSpec((nb, K, c), lambda i: (i, 0, 0))
    v = lambda c: pl.BlockSpec((1, c), lambda i: (0, 0))
    w = lambda a, b_: pl.BlockSpec((a, b_), lambda i: (0, 0))
    return pl.pallas_call(
        _comb2_body,
        grid=(grid,),
        in_specs=[b3(ca), v(ca), v(ca), b3(cb), v(cb), v(cb),
                  w(ca, cout), w(cb, cout), v(cout)],
        out_specs=[b3(cout), b3(cb), pl.BlockSpec((2, cout), lambda i: (0, 0))],
        out_shape=[jax.ShapeDtypeStruct((BN, K, cout), BF16),
                   jax.ShapeDtypeStruct((BN, K, cb), BF16),
                   jax.ShapeDtypeStruct((2, cout), F32)],
    )(ya, sa, ta, yb, sb, tb, Wa, Wb, b)


def _addres_body(ya_ref, sa_ref, ta_ref, res_ref, wa_ref, y_ref, acc_ref):
    nb, K, _ = ya_ref.shape
    xa = jnp.maximum(ya_ref[...] * sa_ref[...] + ta_ref[...], 0.0)
    xa2 = xa.reshape(nb * K, -1)
    y2 = (_dot(xa2, wa_ref[...])
          + res_ref[...].reshape(nb * K, -1).astype(F32))
    y_ref[...] = y2.reshape(y_ref.shape).astype(y_ref.dtype)

    @pl.when(pl.program_id(0) == 0)
    def _():
        acc_ref[...] = jnp.zeros_like(acc_ref)

    acc_ref[...] += jnp.concatenate(
        [jnp.sum(y2, 0, keepdims=True), jnp.sum(y2 * y2, 0, keepdims=True)], 0)


def _addres_pass(ya, sa, ta, res, Wa, K):
    BN = ya.shape[0]
    ca, cout = Wa.shape
    nb = RB // K
    grid = BN // nb
    b3 = lambda c: pl.BlockSpec((nb, K, c), lambda i: (i, 0, 0))
    v = lambda c: pl.BlockSpec((1, c), lambda i: (0, 0))
    return pl.pallas_call(
        _addres_body,
        grid=(grid,),
        in_specs=[b3(ca), v(ca), v(ca), b3(cout),
                  pl.BlockSpec((ca, cout), lambda i: (0, 0))],
        out_specs=[b3(cout), pl.BlockSpec((2, cout), lambda i: (0, 0))],
        out_shape=[jax.ShapeDtypeStruct((BN, K, cout), BF16),
                   jax.ShapeDtypeStruct((2, cout), F32)],
    )(ya, sa, ta, res, Wa)


def _smr_body(y_ref, s_ref, t_ref, v_ref, o_ref, *, vlo):
    c = y_ref.shape[-1]
    x = jnp.maximum(y_ref[...] * s_ref[...] + t_ref[...], 0.0)
    m = jnp.max(x, axis=1, keepdims=True)
    e = jnp.exp(x - m)
    w = e / jnp.sum(e, axis=1, keepdims=True)
    o_ref[...] = jnp.sum(w * v_ref[...][:, :, vlo:vlo + c], axis=1)


def _smr_pass(y, s, t, vals, K, vlo=0):
    BN, _, c = y.shape
    cv = vals.shape[-1]
    nb = RB // K
    grid = BN // nb
    b3 = lambda cc: pl.BlockSpec((nb, K, cc), lambda i: (i, 0, 0))
    v = lambda cc: pl.BlockSpec((1, cc), lambda i: (0, 0))
    return pl.pallas_call(
        functools.partial(_smr_body, vlo=vlo),
        grid=(grid,),
        in_specs=[b3(c), v(c), v(c), b3(cv)],
        out_specs=pl.BlockSpec((nb, c), lambda i: (i, 0)),
        out_shape=jax.ShapeDtypeStruct((BN, c), F32),
    )(y, s, t, vals)


QB = 1024

_INF = 3.0e38


def _knn_body(q_ref, kt_ref, out_ref, d_ref, *, k, n):
    q = q_ref[0]
    kt = kt_ref[0]
    qb = q.astype(jnp.bfloat16).astype(F32)
    ktb = kt.astype(jnp.bfloat16).astype(F32)
    dot = (qb[:, 0:1] * ktb[0:1, :] + qb[:, 1:2] * ktb[1:2, :]
           + qb[:, 2:3] * ktb[2:3, :])
    q2 = jnp.sum(q * q, axis=1, keepdims=True)
    k2 = jnp.sum(kt * kt, axis=0, keepdims=True)
    d_ref[...] = (-2.0 * dot + q2) + k2
    iota_n = jax.lax.broadcasted_iota(jnp.int32, (QB, n), 1)
    iota_k = jax.lax.broadcasted_iota(jnp.int32, (QB, k), 1)

    def body(j, acc):
        dc = d_ref[...]
        m = jnp.min(dc, axis=1, keepdims=True)
        am = jnp.min(jnp.where(dc <= m, iota_n, n), axis=1, keepdims=True)
        d_ref[...] = jnp.where(iota_n == am, _INF, dc)
        return jnp.where(iota_k == j, am, acc)

    out_ref[0] = jax.lax.fori_loop(0, k, body, jnp.zeros((QB, k), jnp.int32))


def _knn(keys, queries, k):
    B, N, _ = keys.shape
    kt = keys.transpose(0, 2, 1)
    return pl.pallas_call(
        functools.partial(_knn_body, k=k, n=N),
        grid=(B, N // QB),
        in_specs=[pl.BlockSpec((1, QB, 3), lambda b, i: (b, i, 0)),
                  pl.BlockSpec((1, 3, N), lambda b, i: (b, 0, 0))],
        out_specs=pl.BlockSpec((1, QB, k), lambda b, i: (b, i, 0)),
        out_shape=jax.ShapeDtypeStruct((B, N, k), jnp.int32),
        scratch_shapes=[pltpu.VMEM((QB, N), F32)],
    )(queries, kt)


def _group(p, idx):
    return jax.vmap(lambda pp, ii: pp[ii])(p, idx)


_CH = 128


def _sc_gather(table, idx_flat):
    from jax.experimental.pallas import tpu_sc as plsc

    M = idx_flat.shape[0]
    NW = 32
    per_w = M // NW
    nch = per_w // _CH
    idx2d = idx_flat.reshape(M // _CH, _CH)
    mesh = plsc.VectorSubcoreMesh(core_axis_name="c", subcore_axis_name="s")

    @functools.partial(
        pl.kernel, mesh=mesh,
        out_type=jax.ShapeDtypeStruct((M, 128), F32),
        scratch_types=[pltpu.VMEM((nch, _CH), jnp.int32),
                       pltpu.VMEM((_CH, 128), F32),
                       pltpu.VMEM((_CH, 128), F32),
                       pltpu.SemaphoreType.DMA,
                       pltpu.SemaphoreType.DMA],
    )
    def k(tab_hbm, idx_hbm, out_hbm, idx_v, buf0, buf1, sem0, sem1):
        wid = jax.lax.axis_index("s") * 2 + jax.lax.axis_index("c")
        base = wid * per_w
        pltpu.sync_copy(idx_hbm.at[pl.ds(wid * nch, nch)], idx_v)
        pltpu.async_copy(tab_hbm.at[idx_v.at[0]], buf0, sem0)

        def body(i, carry):
            j0 = 2 * i
            pltpu.async_copy(tab_hbm.at[idx_v.at[j0 + 1]], buf1, sem1)
            pltpu.make_async_copy(tab_hbm.at[idx_v.at[j0]], buf0, sem0).wait()
            pltpu.sync_copy(buf0, out_hbm.at[pl.ds(base + j0 * _CH, _CH)])

            @pl.when(i + 1 < nch // 2)
            def _():
                pltpu.async_copy(tab_hbm.at[idx_v.at[j0 + 2]], buf0, sem0)

            pltpu.make_async_copy(tab_hbm.at[idx_v.at[j0 + 1]], buf1, sem1).wait()
            pltpu.sync_copy(buf1, out_hbm.at[pl.ds(base + (j0 + 1) * _CH, _CH)])
            return carry

        jax.lax.fori_loop(0, nch // 2, body, 0)

    return k(table, idx2d)


def kernel(warped_xyz, warped_points, f2_xyz, f2_points, params):
    B, N, C = warped_points.shape
    BN = B * N
    Kq, Ks = 32, 16
    M1 = float(BN * Kq)
    M2 = float(BN * Ks)

    (W1, b1, g1, be1), (W2, b2, g2, be2), (W3, b3, g3, be3) = params['mlp1']
    We, be_, ge, bee = params['pi_enc']
    (W4, b4, g4, be4), (W5, b5, g5, be5) = params['mlp2']
    We2, be2_, ge2, bee2 = params['pc_enc']
    (W8, b8, g8, be8), (W9, b9, g9, be9) = params['mlp2_2']

    A_wx, A_gx, A_df = W1[:, 0:3], W1[:, 3:6], W1[:, 6:9]
    A_eu, A_wp, A_gp = W1[:, 9:10], W1[:, 10:10 + C], W1[:, 10 + C:10 + 2 * C]
    E_wx, E_gx, E_df, E_eu = We[:, 0:3], We[:, 3:6], We[:, 6:9], We[:, 9:10]
    E2_wx, E2_gx, E2_df, E2_eu = We2[:, 0:3], We2[:, 3:6], We2[:, 6:9], We2[:, 9:10]
    W8a, W8m, W8g = W8[:, 0:C], W8[:, C:2 * C], W8[:, 2 * C:3 * C]

    zc3 = jnp.zeros((3, 128), F32)
    z64_3 = jnp.zeros((C, 64), F32)
    Wc = jnp.concatenate([
        jnp.concatenate([(A_wx - A_df).T, A_wp.T], 0),
        jnp.concatenate([(E_wx - E_df).T, z64_3], 0),
        jnp.concatenate([(E2_wx - E2_df).T, z64_3], 0),
        jnp.concatenate([zc3, W8m.T], 0),
    ], 1)
    bc = jnp.concatenate([b1, be_, be2_, b8])[None, :]
    cw = jnp.concatenate([warped_xyz, warped_points], -1).reshape(BN, 3 + C)
    one67 = jnp.ones((1, 3 + C), F32)
    zero67 = jnp.zeros((1, 3 + C), F32)
    c_all, _ = _mm_pass(cw, one67, zero67, Wc, bc, None, pre=False, out_dtype=F32)
    c1, c1e = c_all[:, 0:128], c_all[:, 128:192]
    c7e, c7a = c_all[:, 192:256], c_all[:, 256:384]

    off = (jnp.arange(B, dtype=jnp.int32) * N)[:, None, None]
    pad13 = jnp.zeros((B, N, 13), F32)
    pad48 = jnp.zeros((B, N, 48), F32)
    idx1 = _knn(f2_xyz, warped_xyz, Kq)
    tab1 = jnp.concatenate([f2_xyz, pad13, f2_points, pad48], -1).reshape(BN, 128)
    gg1 = _sc_gather(tab1, (idx1 + off).reshape(BN * Kq)).reshape(BN, Kq, 128)
    idx2 = _knn(warped_xyz, warped_xyz, Ks)
    wxyz2 = warped_xyz.reshape(BN, 3)

    Wm1 = jnp.zeros((128, 128), F32).at[0:3].set((A_gx + A_df).T) \
        .at[16:16 + C].set(A_gp.T)
    We1 = jnp.zeros((128, 64), F32).at[0:3].set((E_gx + E_df).T)
    y1, yenc, acc1, acce = _feat_pass(gg1, wxyz2, c1, c1e,
                                      Wm1, A_eu.T, We1, E_eu.T, Kq)

    s1, t1 = _st(acc1, M1, g1, be1)
    y2, acc2 = _mm_pass(y1, s1, t1, W2.T, b2[None], Kq)
    s2, t2 = _st(acc2, M1, g2, be2)
    y3, acc3 = _mm_pass(y2, s2, t2, W3.T, b3[None], Kq)
    s3, t3 = _st(acc3, M1, g3, be3)
    se, te = _st(acce, M1, ge, bee)
    y4, feat, acc4 = _comb2_pass(yenc, se, te, y3, s3, t3,
                                 W4[:, 0:C].T, W4[:, C:2 * C].T, b4[None], Kq)
    s4, t4 = _st(acc4, M1, g4, be4)
    y5, acc5 = _mm_pass(y4, s4, t4, W5.T, b5[None], Kq)
    s5, t5 = _st(acc5, M1, g5, be5)
    out1 = _smr_pass(y5, s5, t5, feat, Kq)

    tab2 = jnp.concatenate(
        [warped_xyz.reshape(BN, 3), pad13.reshape(BN, 13), out1,
         pad48.reshape(BN, 48)], -1)
    gg2 = _sc_gather(tab2, (idx2 + off).reshape(BN * Ks)).reshape(BN, Ks, 128)

    z1_128 = jnp.zeros((1, 128), F32)
    Wm2 = jnp.zeros((128, 128), F32).at[16:16 + C].set(W8g.T)
    We2p = jnp.zeros((128, 64), F32).at[0:3].set((E2_gx + E2_df).T)
    ya, yenc2, _, acce2 = _feat_pass(gg2, wxyz2, c7a, c7e,
                                     Wm2, z1_128, We2p, E2_eu.T, Ks)

    se2, te2 = _st(acce2, M2, ge2, bee2)
    y8, acc8 = _addres_pass(yenc2, se2, te2, ya, W8a.T, Ks)
    s8, t8 = _st(acc8, M2, g8, be8)
    y9, acc9 = _mm_pass(y8, s8, t8, W9.T, b9[None], Ks)
    s9, t9 = _st(acc9, M2, g9, be9)
    out = _smr_pass(y9, s9, t9, gg2, Ks, vlo=16)
    return out.reshape(B, N, C)
